# baseline (speedup 1.0000x reference)
import functools

import jax
import jax.numpy as jnp
from jax import lax
from jax.experimental import pallas as pl
from jax.experimental.pallas import tpu as pltpu
from jax.experimental.pallas import tpu_sc as plsc

_NUM_EXPERTS = 8
_N_TOK = 32768
_D_IN = 768
_D_OUT = 128
_CAPACITY = _N_TOK * 2 // _NUM_EXPERTS

_LOGIT_BLK = 2048
_MOE_BLK = 1024

_N_TILES = 16
_TOK_PER_TILE = _N_TOK // _N_TILES
_GROUPS = _TOK_PER_TILE // 16

_EPS_CACHE = []


def _gating_eps_t():
    if not _EPS_CACHE:
        _EPS_CACHE.append(jax.random.normal(
            jax.random.key(42), (_N_TOK, _NUM_EXPERTS),
            dtype=jnp.float32).T)
    return _EPS_CACHE[0]


def _logits_kernel(x_ref, wgn_ref, eps_ref, noisy_ref):
    logits2 = jnp.dot(x_ref[...], wgn_ref[...],
                      preferred_element_type=jnp.float32)
    lt = logits2.T
    clean = lt[:_NUM_EXPERTS, :]
    raw = lt[_NUM_EXPERTS:, :]
    std = (jnp.maximum(raw, 0.0)
           + jnp.log1p(jnp.exp(-jnp.abs(raw))) + 1e-2)
    noisy_ref[...] = clean + eps_ref[...] * std


def _routing_body(noisy_hbm, gates_hbm, cnt_hbm, chunk, outbuf, g1b, g2b,
                  i1b, i2b, cntb, allcnt):
    wid = lax.axis_index("s")
    base = wid * _TOK_PER_TILE
    pltpu.sync_copy(noisy_hbm.at[:, pl.ds(base, _TOK_PER_TILE)], chunk)
    lanes = jnp.arange(16, dtype=jnp.int32)
    neg_inf = jnp.full((16,), -jnp.inf, dtype=jnp.float32)

    def pass_a(t, accs):
        sl = pl.ds(t * 16, 16)
        vs = [chunk[e, sl] for e in range(_NUM_EXPERTS)]
        m1 = vs[0]
        for e in range(1, _NUM_EXPERTS):
            m1 = jnp.maximum(m1, vs[e])
        i1 = jnp.full((16,), _NUM_EXPERTS, jnp.int32)
        for e in reversed(range(_NUM_EXPERTS)):
            i1 = jnp.where(vs[e] == m1, jnp.int32(e), i1)
        masked = [jnp.where(i1 == e, neg_inf, vs[e])
                  for e in range(_NUM_EXPERTS)]
        m2 = masked[0]
        for e in range(1, _NUM_EXPERTS):
            m2 = jnp.maximum(m2, masked[e])
        i2 = jnp.full((16,), _NUM_EXPERTS, jnp.int32)
        for e in reversed(range(_NUM_EXPERTS)):
            i2 = jnp.where(masked[e] == m2, jnp.int32(e), i2)
        e2 = jnp.exp(m2 - m1)
        denom = 1.0 + e2
        g1 = 1.0 / denom
        g2 = e2 / denom
        g1b[sl] = g1
        g2b[sl] = g2
        i1b[sl] = i1
        i2b[sl] = i2
        g2pos = g2 > 0.0
        out = []
        for e in range(_NUM_EXPERTS):
            mask_e = (i1 == e) | ((i2 == e) & g2pos)
            out.append(accs[e] + mask_e.astype(jnp.int32))
        return tuple(out)

    accs = lax.fori_loop(
        0, _GROUPS, pass_a,
        tuple(jnp.zeros((16,), jnp.int32) for _ in range(_NUM_EXPERTS)))

    cnt_vec = jnp.zeros((16,), jnp.int32)
    for e in range(_NUM_EXPERTS):
        c_e = jnp.sum(accs[e])
        cnt_vec = cnt_vec + jnp.where(lanes == e, c_e, 0)
    cntb[...] = cnt_vec
    pltpu.sync_copy(cntb, cnt_hbm.at[wid])
    plsc.subcore_barrier()

    pltpu.sync_copy(cnt_hbm, allcnt)
    prefix = jnp.zeros((16,), jnp.int32)
    for w in range(_N_TILES):
        row = allcnt[w, :]
        prefix = prefix + jnp.where(w < wid, row, 0)

    def pass_b(t, rcs):
        sl = pl.ds(t * 16, 16)
        g1 = g1b[sl]
        g2 = g2b[sl]
        i1 = i1b[sl]
        i2 = i2b[sl]
        g2pos = g2 > 0.0
        out = []
        for e in range(_NUM_EXPERTS):
            mask_e = (i1 == e) | ((i2 == e) & g2pos)
            mi = mask_e.astype(jnp.int32)
            scan = plsc.cumsum(mi)
            pos = scan + rcs[e]
            keep = mask_e & (pos <= _CAPACITY)
            ge = (jnp.where(i1 == e, g1, 0.0)
                  + jnp.where(i2 == e, g2, 0.0))
            outbuf[e, sl] = jnp.where(keep, ge, 0.0)
            out.append(rcs[e] + jnp.sum(mi))
        return tuple(out)

    lax.fori_loop(0, _GROUPS, pass_b,
                  tuple(prefix[e] for e in range(_NUM_EXPERTS)))
    pltpu.sync_copy(outbuf, gates_hbm.at[:, pl.ds(base, _TOK_PER_TILE)])


_routing_sc = functools.partial(
    pl.kernel,
    out_type=(jax.ShapeDtypeStruct((_NUM_EXPERTS, _N_TOK), jnp.float32),
              jax.ShapeDtypeStruct((_N_TILES, 16), jnp.int32)),
    mesh=plsc.VectorSubcoreMesh(core_axis_name="c", subcore_axis_name="s",
                                num_cores=1),
    scratch_types=[
        pltpu.VMEM((_NUM_EXPERTS, _TOK_PER_TILE), jnp.float32),
        pltpu.VMEM((_NUM_EXPERTS, _TOK_PER_TILE), jnp.float32),
        pltpu.VMEM((_TOK_PER_TILE,), jnp.float32),
        pltpu.VMEM((_TOK_PER_TILE,), jnp.float32),
        pltpu.VMEM((_TOK_PER_TILE,), jnp.int32),
        pltpu.VMEM((_TOK_PER_TILE,), jnp.int32),
        pltpu.VMEM((16,), jnp.int32),
        pltpu.VMEM((_N_TILES, 16), jnp.int32),
    ],
    compiler_params=pltpu.CompilerParams(needs_layout_passes=False),
)(_routing_body)


def _moe_kernel(x_ref, gates_ref, w_ref, b_ref, out_ref):
    xb = x_ref[...].astype(jnp.bfloat16)
    y = jnp.dot(xb, w_ref[...], preferred_element_type=jnp.float32)
    g = gates_ref[...].T
    acc = jnp.dot(g, b_ref[...], preferred_element_type=jnp.float32)
    for e in range(_NUM_EXPERTS):
        acc = acc + y[:, e * _D_OUT:(e + 1) * _D_OUT] * g[:, e:e + 1]
    out_ref[...] = acc


@jax.jit
def kernel(x, W_gate, W_noise, W_experts, b_experts):
    wgn = jnp.concatenate([W_gate, W_noise], axis=1)
    eps_t = _gating_eps_t()

    noisy_t = pl.pallas_call(
        _logits_kernel,
        grid=(_N_TOK // _LOGIT_BLK,),
        in_specs=[
            pl.BlockSpec((_LOGIT_BLK, _D_IN), lambda b: (b, 0)),
            pl.BlockSpec((_D_IN, 2 * _NUM_EXPERTS), lambda b: (0, 0)),
            pl.BlockSpec((_NUM_EXPERTS, _LOGIT_BLK), lambda b: (0, b)),
        ],
        out_specs=pl.BlockSpec((_NUM_EXPERTS, _LOGIT_BLK), lambda b: (0, b)),
        out_shape=jax.ShapeDtypeStruct((_NUM_EXPERTS, _N_TOK), jnp.float32),
        compiler_params=pltpu.CompilerParams(
            dimension_semantics=("arbitrary",)),
    )(x, wgn, eps_t)

    gates_t, _ = _routing_sc(noisy_t)

    w_all = jnp.transpose(W_experts, (1, 0, 2)).reshape(
        _D_IN, _NUM_EXPERTS * _D_OUT).astype(jnp.bfloat16)

    out = pl.pallas_call(
        _moe_kernel,
        grid=(_N_TOK // _MOE_BLK,),
        in_specs=[
            pl.BlockSpec((_MOE_BLK, _D_IN), lambda b: (b, 0)),
            pl.BlockSpec((_NUM_EXPERTS, _MOE_BLK), lambda b: (0, b)),
            pl.BlockSpec((_D_IN, _NUM_EXPERTS * _D_OUT), lambda b: (0, 0)),
            pl.BlockSpec((_NUM_EXPERTS, _D_OUT), lambda b: (0, 0)),
        ],
        out_specs=pl.BlockSpec((_MOE_BLK, _D_OUT), lambda b: (b, 0)),
        out_shape=jax.ShapeDtypeStruct((_N_TOK, _D_OUT), jnp.float32),
        compiler_params=pltpu.CompilerParams(
            dimension_semantics=("arbitrary",)),
    )(x, gates_t, w_all, b_experts)
    return out


# speedup vs baseline: 1.0018x; 1.0018x over previous
import functools

import jax
import jax.numpy as jnp
from jax import lax
from jax.experimental import pallas as pl
from jax.experimental.pallas import tpu as pltpu
from jax.experimental.pallas import tpu_sc as plsc

_NUM_EXPERTS = 8
_N_TOK = 32768
_D_IN = 768
_D_OUT = 128
_CAPACITY = _N_TOK * 2 // _NUM_EXPERTS

_LOGIT_BLK = 4096
_MOE_BLK = 1024

_N_TILES = 16
_TOK_PER_TILE = _N_TOK // _N_TILES
_GROUPS = _TOK_PER_TILE // 16

_EPS_CACHE = []


def _gating_eps_t():
    if not _EPS_CACHE:
        _EPS_CACHE.append(jax.random.normal(
            jax.random.key(42), (_N_TOK, _NUM_EXPERTS),
            dtype=jnp.float32).T)
    return _EPS_CACHE[0]


def _logits_kernel(x_ref, wgn_ref, eps_ref, noisy_ref):
    logits2 = jnp.dot(x_ref[...], wgn_ref[...],
                      preferred_element_type=jnp.float32)
    lt = logits2.T
    clean = lt[:_NUM_EXPERTS, :]
    raw = lt[_NUM_EXPERTS:, :]
    std = (jnp.maximum(raw, 0.0)
           + jnp.log1p(jnp.exp(-jnp.abs(raw))) + 1e-2)
    noisy_ref[...] = clean + eps_ref[...] * std


def _routing_body(noisy_hbm, gates_hbm, cnt_hbm, chunk, outbuf, g1b, g2b,
                  i1b, i2b, cntb, allcnt):
    wid = lax.axis_index("s")
    base = wid * _TOK_PER_TILE
    pltpu.sync_copy(noisy_hbm.at[:, pl.ds(base, _TOK_PER_TILE)], chunk)
    lanes = jnp.arange(16, dtype=jnp.int32)
    neg_inf = jnp.full((16,), -jnp.inf, dtype=jnp.float32)

    def pass_a(t, accs):
        sl = pl.ds(t * 16, 16)
        vs = [chunk[e, sl] for e in range(_NUM_EXPERTS)]
        m1 = vs[0]
        for e in range(1, _NUM_EXPERTS):
            m1 = jnp.maximum(m1, vs[e])
        i1 = jnp.full((16,), _NUM_EXPERTS, jnp.int32)
        for e in reversed(range(_NUM_EXPERTS)):
            i1 = jnp.where(vs[e] == m1, jnp.int32(e), i1)
        masked = [jnp.where(i1 == e, neg_inf, vs[e])
                  for e in range(_NUM_EXPERTS)]
        m2 = masked[0]
        for e in range(1, _NUM_EXPERTS):
            m2 = jnp.maximum(m2, masked[e])
        i2 = jnp.full((16,), _NUM_EXPERTS, jnp.int32)
        for e in reversed(range(_NUM_EXPERTS)):
            i2 = jnp.where(masked[e] == m2, jnp.int32(e), i2)
        e2 = jnp.exp(m2 - m1)
        denom = 1.0 + e2
        g1 = 1.0 / denom
        g2 = e2 / denom
        g1b[sl] = g1
        g2b[sl] = g2
        i1b[sl] = i1
        i2b[sl] = i2
        g2pos = g2 > 0.0
        out = []
        for e in range(_NUM_EXPERTS):
            mask_e = (i1 == e) | ((i2 == e) & g2pos)
            out.append(accs[e] + mask_e.astype(jnp.int32))
        return tuple(out)

    accs = lax.fori_loop(
        0, _GROUPS, pass_a,
        tuple(jnp.zeros((16,), jnp.int32) for _ in range(_NUM_EXPERTS)))

    cnt_vec = jnp.zeros((16,), jnp.int32)
    for e in range(_NUM_EXPERTS):
        c_e = jnp.sum(accs[e])
        cnt_vec = cnt_vec + jnp.where(lanes == e, c_e, 0)
    cntb[...] = cnt_vec
    pltpu.sync_copy(cntb, cnt_hbm.at[wid])
    plsc.subcore_barrier()

    pltpu.sync_copy(cnt_hbm, allcnt)
    prefix = jnp.zeros((16,), jnp.int32)
    for w in range(_N_TILES):
        row = allcnt[w, :]
        prefix = prefix + jnp.where(w < wid, row, 0)

    def pass_b(t, rcs):
        sl = pl.ds(t * 16, 16)
        g1 = g1b[sl]
        g2 = g2b[sl]
        i1 = i1b[sl]
        i2 = i2b[sl]
        g2pos = g2 > 0.0
        out = []
        for e in range(_NUM_EXPERTS):
            mask_e = (i1 == e) | ((i2 == e) & g2pos)
            mi = mask_e.astype(jnp.int32)
            scan = plsc.cumsum(mi)
            pos = scan + rcs[e]
            keep = mask_e & (pos <= _CAPACITY)
            ge = (jnp.where(i1 == e, g1, 0.0)
                  + jnp.where(i2 == e, g2, 0.0))
            outbuf[e, sl] = jnp.where(keep, ge, 0.0)
            out.append(rcs[e] + jnp.sum(mi))
        return tuple(out)

    lax.fori_loop(0, _GROUPS, pass_b,
                  tuple(prefix[e] for e in range(_NUM_EXPERTS)))
    pltpu.sync_copy(outbuf, gates_hbm.at[:, pl.ds(base, _TOK_PER_TILE)])


_routing_sc = functools.partial(
    pl.kernel,
    out_type=(jax.ShapeDtypeStruct((_NUM_EXPERTS, _N_TOK), jnp.float32),
              jax.ShapeDtypeStruct((_N_TILES, 16), jnp.int32)),
    mesh=plsc.VectorSubcoreMesh(core_axis_name="c", subcore_axis_name="s",
                                num_cores=1),
    scratch_types=[
        pltpu.VMEM((_NUM_EXPERTS, _TOK_PER_TILE), jnp.float32),
        pltpu.VMEM((_NUM_EXPERTS, _TOK_PER_TILE), jnp.float32),
        pltpu.VMEM((_TOK_PER_TILE,), jnp.float32),
        pltpu.VMEM((_TOK_PER_TILE,), jnp.float32),
        pltpu.VMEM((_TOK_PER_TILE,), jnp.int32),
        pltpu.VMEM((_TOK_PER_TILE,), jnp.int32),
        pltpu.VMEM((16,), jnp.int32),
        pltpu.VMEM((_N_TILES, 16), jnp.int32),
    ],
    compiler_params=pltpu.CompilerParams(needs_layout_passes=False),
)(_routing_body)


def _moe_kernel(x_ref, gates_ref, w_ref, b_ref, out_ref):
    xb = x_ref[...].astype(jnp.bfloat16)
    y = jnp.dot(xb, w_ref[...], preferred_element_type=jnp.float32)
    g = gates_ref[...].T
    acc = jnp.dot(g, b_ref[...], preferred_element_type=jnp.float32)
    for e in range(_NUM_EXPERTS):
        acc = acc + y[:, e * _D_OUT:(e + 1) * _D_OUT] * g[:, e:e + 1]
    out_ref[...] = acc


@jax.jit
def kernel(x, W_gate, W_noise, W_experts, b_experts):
    wgn = jnp.concatenate([W_gate, W_noise], axis=1)
    eps_t = _gating_eps_t()

    noisy_t = pl.pallas_call(
        _logits_kernel,
        grid=(_N_TOK // _LOGIT_BLK,),
        in_specs=[
            pl.BlockSpec((_LOGIT_BLK, _D_IN), lambda b: (b, 0)),
            pl.BlockSpec((_D_IN, 2 * _NUM_EXPERTS), lambda b: (0, 0)),
            pl.BlockSpec((_NUM_EXPERTS, _LOGIT_BLK), lambda b: (0, b)),
        ],
        out_specs=pl.BlockSpec((_NUM_EXPERTS, _LOGIT_BLK), lambda b: (0, b)),
        out_shape=jax.ShapeDtypeStruct((_NUM_EXPERTS, _N_TOK), jnp.float32),
        compiler_params=pltpu.CompilerParams(
            dimension_semantics=("arbitrary",)),
    )(x, wgn, eps_t)

    gates_t, _ = _routing_sc(noisy_t)

    w_all = jnp.transpose(W_experts, (1, 0, 2)).reshape(
        _D_IN, _NUM_EXPERTS * _D_OUT).astype(jnp.bfloat16)

    out = pl.pallas_call(
        _moe_kernel,
        grid=(_N_TOK // _MOE_BLK,),
        in_specs=[
            pl.BlockSpec((_MOE_BLK, _D_IN), lambda b: (b, 0)),
            pl.BlockSpec((_NUM_EXPERTS, _MOE_BLK), lambda b: (0, b)),
            pl.BlockSpec((_D_IN, _NUM_EXPERTS * _D_OUT), lambda b: (0, 0)),
            pl.BlockSpec((_NUM_EXPERTS, _D_OUT), lambda b: (0, 0)),
        ],
        out_specs=pl.BlockSpec((_MOE_BLK, _D_OUT), lambda b: (b, 0)),
        out_shape=jax.ShapeDtypeStruct((_N_TOK, _D_OUT), jnp.float32),
        compiler_params=pltpu.CompilerParams(
            dimension_semantics=("arbitrary",)),
    )(x, gates_t, w_all, b_experts)
    return out


# speedup vs baseline: 1.0080x; 1.0063x over previous
import functools

import jax
import jax.numpy as jnp
from jax import lax
from jax.experimental import pallas as pl
from jax.experimental.pallas import tpu as pltpu
from jax.experimental.pallas import tpu_sc as plsc

_NUM_EXPERTS = 8
_N_TOK = 32768
_D_IN = 768
_D_OUT = 128
_CAPACITY = _N_TOK * 2 // _NUM_EXPERTS

_LOGIT_BLK = 4096
_MOE_BLK = 2048

_N_TILES = 16
_TOK_PER_TILE = _N_TOK // _N_TILES
_GROUPS = _TOK_PER_TILE // 16

_EPS_CACHE = []


def _gating_eps_t():
    if not _EPS_CACHE:
        _EPS_CACHE.append(jax.random.normal(
            jax.random.key(42), (_N_TOK, _NUM_EXPERTS),
            dtype=jnp.float32).T)
    return _EPS_CACHE[0]


def _logits_kernel(x_ref, wgn_ref, eps_ref, noisy_ref):
    logits2 = jnp.dot(x_ref[...], wgn_ref[...],
                      preferred_element_type=jnp.float32)
    lt = logits2.T
    clean = lt[:_NUM_EXPERTS, :]
    raw = lt[_NUM_EXPERTS:, :]
    std = (jnp.maximum(raw, 0.0)
           + jnp.log1p(jnp.exp(-jnp.abs(raw))) + 1e-2)
    noisy_ref[...] = clean + eps_ref[...] * std


def _routing_body(noisy_hbm, gates_hbm, cnt_hbm, chunk, outbuf, g1b, g2b,
                  i1b, i2b, cntb, allcnt):
    wid = lax.axis_index("s")
    base = wid * _TOK_PER_TILE
    pltpu.sync_copy(noisy_hbm.at[:, pl.ds(base, _TOK_PER_TILE)], chunk)
    lanes = jnp.arange(16, dtype=jnp.int32)
    neg_inf = jnp.full((16,), -jnp.inf, dtype=jnp.float32)

    def pass_a(t, accs):
        sl = pl.ds(t * 16, 16)
        vs = [chunk[e, sl] for e in range(_NUM_EXPERTS)]
        m1 = vs[0]
        for e in range(1, _NUM_EXPERTS):
            m1 = jnp.maximum(m1, vs[e])
        i1 = jnp.full((16,), _NUM_EXPERTS, jnp.int32)
        for e in reversed(range(_NUM_EXPERTS)):
            i1 = jnp.where(vs[e] == m1, jnp.int32(e), i1)
        masked = [jnp.where(i1 == e, neg_inf, vs[e])
                  for e in range(_NUM_EXPERTS)]
        m2 = masked[0]
        for e in range(1, _NUM_EXPERTS):
            m2 = jnp.maximum(m2, masked[e])
        i2 = jnp.full((16,), _NUM_EXPERTS, jnp.int32)
        for e in reversed(range(_NUM_EXPERTS)):
            i2 = jnp.where(masked[e] == m2, jnp.int32(e), i2)
        e2 = jnp.exp(m2 - m1)
        denom = 1.0 + e2
        g1 = 1.0 / denom
        g2 = e2 / denom
        g1b[sl] = g1
        g2b[sl] = g2
        i1b[sl] = i1
        i2b[sl] = i2
        g2pos = g2 > 0.0
        out = []
        for e in range(_NUM_EXPERTS):
            mask_e = (i1 == e) | ((i2 == e) & g2pos)
            out.append(accs[e] + mask_e.astype(jnp.int32))
        return tuple(out)

    accs = lax.fori_loop(
        0, _GROUPS, pass_a,
        tuple(jnp.zeros((16,), jnp.int32) for _ in range(_NUM_EXPERTS)))

    cnt_vec = jnp.zeros((16,), jnp.int32)
    for e in range(_NUM_EXPERTS):
        c_e = jnp.sum(accs[e])
        cnt_vec = cnt_vec + jnp.where(lanes == e, c_e, 0)
    cntb[...] = cnt_vec
    pltpu.sync_copy(cntb, cnt_hbm.at[wid])
    plsc.subcore_barrier()

    pltpu.sync_copy(cnt_hbm, allcnt)
    prefix = jnp.zeros((16,), jnp.int32)
    for w in range(_N_TILES):
        row = allcnt[w, :]
        prefix = prefix + jnp.where(w < wid, row, 0)

    def pass_b(t, rcs):
        sl = pl.ds(t * 16, 16)
        g1 = g1b[sl]
        g2 = g2b[sl]
        i1 = i1b[sl]
        i2 = i2b[sl]
        g2pos = g2 > 0.0
        out = []
        for e in range(_NUM_EXPERTS):
            mask_e = (i1 == e) | ((i2 == e) & g2pos)
            mi = mask_e.astype(jnp.int32)
            scan = plsc.cumsum(mi)
            pos = scan + rcs[e]
            keep = mask_e & (pos <= _CAPACITY)
            ge = (jnp.where(i1 == e, g1, 0.0)
                  + jnp.where(i2 == e, g2, 0.0))
            outbuf[e, sl] = jnp.where(keep, ge, 0.0)
            out.append(rcs[e] + jnp.sum(mi))
        return tuple(out)

    lax.fori_loop(0, _GROUPS, pass_b,
                  tuple(prefix[e] for e in range(_NUM_EXPERTS)))
    pltpu.sync_copy(outbuf, gates_hbm.at[:, pl.ds(base, _TOK_PER_TILE)])


_routing_sc = functools.partial(
    pl.kernel,
    out_type=(jax.ShapeDtypeStruct((_NUM_EXPERTS, _N_TOK), jnp.float32),
              jax.ShapeDtypeStruct((_N_TILES, 16), jnp.int32)),
    mesh=plsc.VectorSubcoreMesh(core_axis_name="c", subcore_axis_name="s",
                                num_cores=1),
    scratch_types=[
        pltpu.VMEM((_NUM_EXPERTS, _TOK_PER_TILE), jnp.float32),
        pltpu.VMEM((_NUM_EXPERTS, _TOK_PER_TILE), jnp.float32),
        pltpu.VMEM((_TOK_PER_TILE,), jnp.float32),
        pltpu.VMEM((_TOK_PER_TILE,), jnp.float32),
        pltpu.VMEM((_TOK_PER_TILE,), jnp.int32),
        pltpu.VMEM((_TOK_PER_TILE,), jnp.int32),
        pltpu.VMEM((16,), jnp.int32),
        pltpu.VMEM((_N_TILES, 16), jnp.int32),
    ],
    compiler_params=pltpu.CompilerParams(needs_layout_passes=False),
)(_routing_body)


def _moe_kernel(x_ref, gates_ref, w_ref, b_ref, out_ref):
    xb = x_ref[...].astype(jnp.bfloat16)
    y = jnp.dot(xb, w_ref[...], preferred_element_type=jnp.float32)
    g = gates_ref[...].T
    acc = jnp.dot(g, b_ref[...], preferred_element_type=jnp.float32)
    for e in range(_NUM_EXPERTS):
        acc = acc + y[:, e * _D_OUT:(e + 1) * _D_OUT] * g[:, e:e + 1]
    out_ref[...] = acc


@jax.jit
def kernel(x, W_gate, W_noise, W_experts, b_experts):
    wgn = jnp.concatenate([W_gate, W_noise], axis=1)
    eps_t = _gating_eps_t()

    noisy_t = pl.pallas_call(
        _logits_kernel,
        grid=(_N_TOK // _LOGIT_BLK,),
        in_specs=[
            pl.BlockSpec((_LOGIT_BLK, _D_IN), lambda b: (b, 0)),
            pl.BlockSpec((_D_IN, 2 * _NUM_EXPERTS), lambda b: (0, 0)),
            pl.BlockSpec((_NUM_EXPERTS, _LOGIT_BLK), lambda b: (0, b)),
        ],
        out_specs=pl.BlockSpec((_NUM_EXPERTS, _LOGIT_BLK), lambda b: (0, b)),
        out_shape=jax.ShapeDtypeStruct((_NUM_EXPERTS, _N_TOK), jnp.float32),
        compiler_params=pltpu.CompilerParams(
            dimension_semantics=("parallel",)),
    )(x, wgn, eps_t)

    gates_t, _ = _routing_sc(noisy_t)

    w_all = jnp.transpose(W_experts, (1, 0, 2)).reshape(
        _D_IN, _NUM_EXPERTS * _D_OUT).astype(jnp.bfloat16)

    out = pl.pallas_call(
        _moe_kernel,
        grid=(_N_TOK // _MOE_BLK,),
        in_specs=[
            pl.BlockSpec((_MOE_BLK, _D_IN), lambda b: (b, 0)),
            pl.BlockSpec((_NUM_EXPERTS, _MOE_BLK), lambda b: (0, b)),
            pl.BlockSpec((_D_IN, _NUM_EXPERTS * _D_OUT), lambda b: (0, 0)),
            pl.BlockSpec((_NUM_EXPERTS, _D_OUT), lambda b: (0, 0)),
        ],
        out_specs=pl.BlockSpec((_MOE_BLK, _D_OUT), lambda b: (b, 0)),
        out_shape=jax.ShapeDtypeStruct((_N_TOK, _D_OUT), jnp.float32),
        compiler_params=pltpu.CompilerParams(
            dimension_semantics=("parallel",)),
    )(x, gates_t, w_all, b_experts)
    return out
